# Initial kernel scaffold; baseline (speedup 1.0000x reference)
import jax
import jax.numpy as jnp
from jax.experimental import pallas as pl


def kernel(node_types, node_values, depth, node_table, value_table, depth_table):
    raise NotImplementedError("write your pallas kernel here")


# speedup vs baseline: 4.0814x; 4.0814x over previous
import functools

import jax
import jax.numpy as jnp
from jax import lax
from jax.experimental import pallas as pl
from jax.experimental.pallas import tpu as pltpu
from jax.experimental.pallas import tpu_sc as plsc

HIDDEN_DIM = 128
MAX_DEPTH = 64
BATCH = 128
MAX_NODES = 256
VALUE_LEN = 32

NUM_CORES = 2
NUM_SUBCORES = 16
NUM_WORKERS = NUM_CORES * NUM_SUBCORES
LANES = 16

TOTAL_ROWS = BATCH * MAX_NODES
ROWS_PER_WORKER = TOTAL_ROWS // NUM_WORKERS
CHUNK = 16
CHUNKS_PER_WORKER = ROWS_PER_WORKER // CHUNK
VCOLS = 128
VROWS_PER_CHUNK = CHUNK * VALUE_LEN // VCOLS


def _body(nt_hbm, dp_hbm, nv_hbm, node_tab, value_tab, depth_tab, out_hbm,
          nidx, didx, vidx, nrows, drows, vrows, obuf, sem):
    wid = lax.axis_index("s") * NUM_CORES + lax.axis_index("c")

    pltpu.sync_copy(nt_hbm.at[pl.ds(wid * (ROWS_PER_WORKER // LANES),
                                    ROWS_PER_WORKER // LANES)], nidx)
    pltpu.sync_copy(dp_hbm.at[pl.ds(wid * (ROWS_PER_WORKER // LANES),
                                    ROWS_PER_WORKER // LANES)], didx)
    pltpu.sync_copy(nv_hbm.at[pl.ds(wid * (ROWS_PER_WORKER * VALUE_LEN // VCOLS),
                                    ROWS_PER_WORKER * VALUE_LEN // VCOLS)], vidx)

    def clamp_body(i, _):
        didx[i, :] = jnp.clip(didx[i, :], 0, MAX_DEPTH - 1)
        return 0
    lax.fori_loop(0, ROWS_PER_WORKER // LANES, clamp_body, 0)

    def chunk_body(c, _):
        base = wid * ROWS_PER_WORKER + c * CHUNK
        d_n = pltpu.async_copy(node_tab.at[nidx.at[c]], nrows, sem)
        d_d = pltpu.async_copy(depth_tab.at[didx.at[c]], drows, sem)
        d_vs = [
            pltpu.async_copy(value_tab.at[vidx.at[c * VROWS_PER_CHUNK + k]],
                             vrows.at[pl.ds(k * VCOLS, VCOLS)], sem)
            for k in range(VROWS_PER_CHUNK)
        ]
        d_n.wait()
        d_d.wait()
        for d in d_vs:
            d.wait()

        nj = HIDDEN_DIM // LANES

        def node_body(n, _):
            row0 = n * VALUE_LEN

            def l_body(l, accs):
                return tuple(
                    accs[j] + vrows[row0 + l, pl.ds(j * LANES, LANES)]
                    for j in range(nj))

            accs = tuple(vrows[row0, pl.ds(j * LANES, LANES)]
                         for j in range(nj))
            accs = lax.fori_loop(1, VALUE_LEN, l_body, accs)
            scale = jnp.float32(1.0 / VALUE_LEN)
            for j in range(nj):
                obuf[n, pl.ds(j * LANES, LANES)] = (
                    accs[j] * scale
                    + nrows[n, pl.ds(j * LANES, LANES)]
                    + drows[n, pl.ds(j * LANES, LANES)])
            return 0

        lax.fori_loop(0, CHUNK, node_body, 0)
        pltpu.sync_copy(obuf, out_hbm.at[pl.ds(base, CHUNK)])
        return 0

    lax.fori_loop(0, CHUNKS_PER_WORKER, chunk_body, 0)


@jax.jit
def _tree_embedding(nt2, dp2, nv2, node_table, value_table, depth_table):
    mesh = plsc.VectorSubcoreMesh(core_axis_name="c", subcore_axis_name="s")
    return pl.kernel(
        _body,
        out_type=jax.ShapeDtypeStruct((TOTAL_ROWS, HIDDEN_DIM), jnp.float32),
        mesh=mesh,
        scratch_types=[
            pltpu.VMEM((ROWS_PER_WORKER // LANES, LANES), jnp.int32),
            pltpu.VMEM((ROWS_PER_WORKER // LANES, LANES), jnp.int32),
            pltpu.VMEM((ROWS_PER_WORKER * VALUE_LEN // VCOLS, VCOLS),
                       jnp.int32),
            pltpu.VMEM((CHUNK, HIDDEN_DIM), jnp.float32),
            pltpu.VMEM((CHUNK, HIDDEN_DIM), jnp.float32),
            pltpu.VMEM((CHUNK * VALUE_LEN, HIDDEN_DIM), jnp.float32),
            pltpu.VMEM((CHUNK, HIDDEN_DIM), jnp.float32),
            pltpu.SemaphoreType.DMA,
        ],
    )(nt2, dp2, nv2, node_table, value_table, depth_table)


def kernel(node_types, node_values, depth, node_table, value_table, depth_table):
    nt2 = node_types.reshape(TOTAL_ROWS // LANES, LANES).astype(jnp.int32)
    dp2 = depth.reshape(TOTAL_ROWS // LANES, LANES).astype(jnp.int32)
    nv2 = node_values.reshape(TOTAL_ROWS * VALUE_LEN // VCOLS,
                              VCOLS).astype(jnp.int32)
    out = _tree_embedding(nt2, dp2, nv2, node_table, value_table, depth_table)
    return out.reshape(BATCH, MAX_NODES, HIDDEN_DIM)
